# baseline (speedup 1.0000x reference)
import functools

import jax
import jax.numpy as jnp
from jax import lax
from jax.experimental import pallas as pl
from jax.experimental.pallas import tpu as pltpu
from jax.experimental.pallas import tpu_sc as plsc

_NC = 2
_NS = 16
_NW = _NC * _NS


def _gather_rows(table, idx):
    B = idx.shape[0]
    D = table.shape[1]
    bpw = B // _NW
    mesh = plsc.VectorSubcoreMesh(core_axis_name="c", subcore_axis_name="s")

    @functools.partial(
        pl.kernel,
        mesh=mesh,
        compiler_params=pltpu.CompilerParams(use_tc_tiling_on_sc=False),
        out_type=jax.ShapeDtypeStruct((B, D), jnp.float32),
        scratch_types=[
            pltpu.VMEM((bpw,), jnp.int32),
            pltpu.VMEM((bpw, D), jnp.float32),
            pltpu.SemaphoreType.DMA,
        ],
    )
    def k(table_hbm, idx_hbm, out_hbm, idx_v, rows_v, sem):
        wid = lax.axis_index("s") * _NC + lax.axis_index("c")
        base = wid * bpw
        pltpu.sync_copy(idx_hbm.at[pl.ds(base, bpw)], idx_v)
        pltpu.async_copy(table_hbm.at[idx_v], rows_v, sem).wait()
        pltpu.sync_copy(rows_v, out_hbm.at[pl.ds(base, bpw)])

    return k(table, idx)


def _mlp_body(x_ref, w1_ref, b1_ref, g1_ref, be1_ref,
              w2_ref, b2_ref, g2_ref, be2_ref, w3_ref, b3_ref, o_ref):
    hi = jax.lax.Precision.HIGHEST
    x = x_ref[...]
    h = lax.dot_general(x, w1_ref[...], (((1,), (0,)), ((), ())),
                        preferred_element_type=jnp.float32, precision=hi)
    h = h + b1_ref[...][None, :]
    mean = jnp.mean(h, axis=0)
    var = jnp.mean(h * h, axis=0) - mean * mean
    s = g1_ref[...] * lax.rsqrt(var + 1e-5)
    h = jnp.maximum(h * s[None, :] + (be1_ref[...] - mean * s)[None, :], 0.0)

    h2 = lax.dot_general(h, w2_ref[...], (((1,), (0,)), ((), ())),
                         preferred_element_type=jnp.float32, precision=hi)
    h2 = h2 + b2_ref[...][None, :]
    mean2 = jnp.mean(h2, axis=0)
    var2 = jnp.mean(h2 * h2, axis=0) - mean2 * mean2
    s2 = g2_ref[...] * lax.rsqrt(var2 + 1e-5)
    h2 = jnp.maximum(h2 * s2[None, :] + (be2_ref[...] - mean2 * s2)[None, :], 0.0)

    o = lax.dot_general(h2, w3_ref[...], (((1,), (0,)), ((), ())),
                        preferred_element_type=jnp.float32, precision=hi)
    o_ref[...] = o + b3_ref[...]


def kernel(x_numeric, x_diag_cat, table, W1, b1, g1, be1, W2, b2, g2, be2, W3, b3):
    B, F = x_numeric.shape
    D = table.shape[1]
    emb = _gather_rows(table, jnp.reshape(x_diag_cat, (B,)))

    K = F + D
    Kp = 32
    x = jnp.concatenate(
        [x_numeric, emb, jnp.zeros((B, Kp - K), jnp.float32)], axis=1)
    w1p = jnp.zeros((Kp, W1.shape[0]), jnp.float32).at[:K, :].set(W1.T)

    out = pl.pallas_call(
        _mlp_body,
        out_shape=jax.ShapeDtypeStruct((B, 1), jnp.float32),
    )(x, w1p, b1, g1, be1, W2.T, b2, g2, be2, W3.T, jnp.reshape(b3, (1, 1)))
    return out


# speedup vs baseline: 1.1026x; 1.1026x over previous
import functools

import jax
import jax.numpy as jnp
from jax import lax
from jax.experimental import pallas as pl
from jax.experimental.pallas import tpu as pltpu
from jax.experimental.pallas import tpu_sc as plsc

_NC = 2
_NS = 16
_NW = _NC * _NS


def _gather_rows(table, idx):
    B = idx.shape[0]
    D = table.shape[1]
    bpw = B // _NW
    mesh = plsc.VectorSubcoreMesh(core_axis_name="c", subcore_axis_name="s")

    @functools.partial(
        pl.kernel,
        mesh=mesh,
        compiler_params=pltpu.CompilerParams(use_tc_tiling_on_sc=False),
        out_type=jax.ShapeDtypeStruct((B, D), jnp.float32),
        scratch_types=[
            pltpu.VMEM((bpw,), jnp.int32),
            pltpu.VMEM((bpw, D), jnp.float32),
            pltpu.SemaphoreType.DMA,
        ],
    )
    def k(table_hbm, idx_hbm, out_hbm, idx_v, rows_v, sem):
        wid = lax.axis_index("s") * _NC + lax.axis_index("c")
        base = wid * bpw
        pltpu.sync_copy(idx_hbm.at[pl.ds(base, bpw)], idx_v)
        pltpu.async_copy(table_hbm.at[idx_v], rows_v, sem).wait()
        pltpu.sync_copy(rows_v, out_hbm.at[pl.ds(base, bpw)])

    return k(table, idx)


def _mlp_body(x_ref, w1_ref, b1_ref, g1_ref, be1_ref,
              w2_ref, b2_ref, g2_ref, be2_ref, w3_ref, b3_ref, o_ref):
    hi = jax.lax.Precision.DEFAULT
    x = x_ref[...]
    h = lax.dot_general(x, w1_ref[...], (((1,), (0,)), ((), ())),
                        preferred_element_type=jnp.float32, precision=hi)
    h = h + b1_ref[...][None, :]
    mean = jnp.mean(h, axis=0)
    var = jnp.mean(h * h, axis=0) - mean * mean
    s = g1_ref[...] * lax.rsqrt(var + 1e-5)
    h = jnp.maximum(h * s[None, :] + (be1_ref[...] - mean * s)[None, :], 0.0)

    h2 = lax.dot_general(h, w2_ref[...], (((1,), (0,)), ((), ())),
                         preferred_element_type=jnp.float32, precision=hi)
    h2 = h2 + b2_ref[...][None, :]
    mean2 = jnp.mean(h2, axis=0)
    var2 = jnp.mean(h2 * h2, axis=0) - mean2 * mean2
    s2 = g2_ref[...] * lax.rsqrt(var2 + 1e-5)
    h2 = jnp.maximum(h2 * s2[None, :] + (be2_ref[...] - mean2 * s2)[None, :], 0.0)

    o = lax.dot_general(h2, w3_ref[...], (((1,), (0,)), ((), ())),
                        preferred_element_type=jnp.float32, precision=hi)
    o_ref[...] = o + b3_ref[...]


def kernel(x_numeric, x_diag_cat, table, W1, b1, g1, be1, W2, b2, g2, be2, W3, b3):
    B, F = x_numeric.shape
    D = table.shape[1]
    emb = _gather_rows(table, jnp.reshape(x_diag_cat, (B,)))

    K = F + D
    Kp = 32
    x = jnp.concatenate(
        [x_numeric, emb, jnp.zeros((B, Kp - K), jnp.float32)], axis=1)
    w1p = jnp.zeros((Kp, W1.shape[0]), jnp.float32).at[:K, :].set(W1.T)

    out = pl.pallas_call(
        _mlp_body,
        out_shape=jax.ShapeDtypeStruct((B, 1), jnp.float32),
    )(x, w1p, b1, g1, be1, W2.T, b2, g2, be2, W3.T, jnp.reshape(b3, (1, 1)))
    return out


# speedup vs baseline: 1.1027x; 1.0001x over previous
import functools

import jax
import jax.numpy as jnp
from jax import lax
from jax.experimental import pallas as pl
from jax.experimental.pallas import tpu as pltpu
from jax.experimental.pallas import tpu_sc as plsc

_NC = 2
_NS = 16
_NW = _NC * _NS


def _gather_rows(table, idx):
    B = idx.shape[0]
    D = table.shape[1]
    bpw = B // _NW
    mesh = plsc.VectorSubcoreMesh(core_axis_name="c", subcore_axis_name="s")

    @functools.partial(
        pl.kernel,
        mesh=mesh,
        compiler_params=pltpu.CompilerParams(use_tc_tiling_on_sc=False),
        out_type=jax.ShapeDtypeStruct((B, D), jnp.float32),
        scratch_types=[
            pltpu.VMEM((bpw,), jnp.int32),
            pltpu.VMEM((bpw, D), jnp.float32),
            pltpu.SemaphoreType.DMA,
        ],
    )
    def k(table_hbm, idx_hbm, out_hbm, idx_v, rows_v, sem):
        wid = lax.axis_index("s") * _NC + lax.axis_index("c")
        base = wid * bpw
        pltpu.sync_copy(idx_hbm.at[pl.ds(base, bpw)], idx_v)
        pltpu.async_copy(table_hbm.at[idx_v], rows_v, sem).wait()
        pltpu.sync_copy(rows_v, out_hbm.at[pl.ds(base, bpw)])

    return k(table, idx)


def _mlp_body(x_ref, w1_ref, b1_ref, g1_ref, be1_ref,
              w2_ref, b2_ref, g2_ref, be2_ref, w3_ref, b3_ref, o_ref):
    hi = jax.lax.Precision.DEFAULT
    x = x_ref[...]
    h = lax.dot_general(x, w1_ref[...], (((1,), (0,)), ((), ())),
                        preferred_element_type=jnp.float32, precision=hi)
    h = h + b1_ref[...][None, :]
    mean = jnp.mean(h, axis=0)
    var = jnp.mean(h * h, axis=0) - mean * mean
    s = g1_ref[...] * lax.rsqrt(var + 1e-5)
    h = jnp.maximum(h * s[None, :] + (be1_ref[...] - mean * s)[None, :], 0.0)

    h2 = lax.dot_general(h, w2_ref[...], (((1,), (0,)), ((), ())),
                         preferred_element_type=jnp.float32, precision=hi)
    h2 = h2 + b2_ref[...][None, :]
    mean2 = jnp.mean(h2, axis=0)
    var2 = jnp.mean(h2 * h2, axis=0) - mean2 * mean2
    s2 = g2_ref[...] * lax.rsqrt(var2 + 1e-5)
    h2 = jnp.maximum(h2 * s2[None, :] + (be2_ref[...] - mean2 * s2)[None, :], 0.0)

    o = jnp.sum(h2 * w3_ref[...], axis=1, keepdims=True)
    o_ref[...] = o + b3_ref[...]


def kernel(x_numeric, x_diag_cat, table, W1, b1, g1, be1, W2, b2, g2, be2, W3, b3):
    B, F = x_numeric.shape
    D = table.shape[1]
    emb = _gather_rows(table, jnp.reshape(x_diag_cat, (B,)))

    K = F + D
    Kp = 32
    x = jnp.concatenate(
        [x_numeric, emb, jnp.zeros((B, Kp - K), jnp.float32)], axis=1)
    w1p = jnp.zeros((Kp, W1.shape[0]), jnp.float32).at[:K, :].set(W1.T)

    out = pl.pallas_call(
        _mlp_body,
        out_shape=jax.ShapeDtypeStruct((B, 1), jnp.float32),
    )(x, w1p, b1, g1, be1, W2.T, b2, g2, be2, jnp.reshape(W3, (1, 128)),
      jnp.reshape(b3, (1, 1)))
    return out


# speedup vs baseline: 1.6514x; 1.4976x over previous
import functools

import jax
import jax.numpy as jnp
from jax import lax
from jax.experimental import pallas as pl
from jax.experimental.pallas import tpu as pltpu
from jax.experimental.pallas import tpu_sc as plsc

_NC = 2
_NS = 16
_NW = _NC * _NS


def _gather_rows(table, idx):
    B = idx.shape[0]
    V, D = table.shape
    bpw = B // _NW
    CH = 16
    NCH = bpw // CH
    mesh = plsc.VectorSubcoreMesh(core_axis_name="c", subcore_axis_name="s")

    @functools.partial(
        pl.kernel,
        mesh=mesh,
        out_type=jax.ShapeDtypeStruct((B, D), jnp.float32),
        scratch_types=[
            pltpu.VMEM((bpw,), jnp.int32),
            pltpu.VMEM((bpw, D), jnp.float32),
            pltpu.SemaphoreType.DMA,
        ],
    )
    def k(table_hbm, idx_hbm, out_hbm, idx_v, rows_v, sem):
        wid = lax.axis_index("s") * _NC + lax.axis_index("c")
        base = wid * bpw
        lanes = lax.iota(jnp.int32, 16)
        pltpu.sync_copy(idx_hbm.at[pl.ds(base, bpw)], idx_v)

        def chunk(c, carry):
            v = idx_v[pl.ds(c * CH, CH)]
            copies = []
            for k_ in range(CH):
                r = v[k_]
                copies.append(pltpu.async_copy(
                    table_hbm.at[r], rows_v.at[c * CH + k_], sem))
            for cp in copies:
                cp.wait()
            return carry

        for c_ in range(NCH):
            chunk(c_, 0)
        pltpu.sync_copy(rows_v, out_hbm.at[pl.ds(base, bpw)])

    return k(table, idx)


def _mlp_body(x_ref, w1_ref, b1_ref, g1_ref, be1_ref,
              w2_ref, b2_ref, g2_ref, be2_ref, w3_ref, b3_ref, o_ref):
    hi = jax.lax.Precision.DEFAULT
    x = x_ref[...]
    h = lax.dot_general(x, w1_ref[...], (((1,), (0,)), ((), ())),
                        preferred_element_type=jnp.float32, precision=hi)
    h = h + b1_ref[...][None, :]
    mean = jnp.mean(h, axis=0)
    var = jnp.mean(h * h, axis=0) - mean * mean
    s = g1_ref[...] * lax.rsqrt(var + 1e-5)
    h = jnp.maximum(h * s[None, :] + (be1_ref[...] - mean * s)[None, :], 0.0)

    h2 = lax.dot_general(h, w2_ref[...], (((1,), (0,)), ((), ())),
                         preferred_element_type=jnp.float32, precision=hi)
    h2 = h2 + b2_ref[...][None, :]
    mean2 = jnp.mean(h2, axis=0)
    var2 = jnp.mean(h2 * h2, axis=0) - mean2 * mean2
    s2 = g2_ref[...] * lax.rsqrt(var2 + 1e-5)
    h2 = jnp.maximum(h2 * s2[None, :] + (be2_ref[...] - mean2 * s2)[None, :], 0.0)

    o = jnp.sum(h2 * w3_ref[...], axis=1, keepdims=True)
    o_ref[...] = o + b3_ref[...]


def kernel(x_numeric, x_diag_cat, table, W1, b1, g1, be1, W2, b2, g2, be2, W3, b3):
    B, F = x_numeric.shape
    D = table.shape[1]
    emb = _gather_rows(table, jnp.reshape(x_diag_cat, (B,)))

    K = F + D
    Kp = 32
    x = jnp.concatenate(
        [x_numeric, emb, jnp.zeros((B, Kp - K), jnp.float32)], axis=1)
    w1p = jnp.zeros((Kp, W1.shape[0]), jnp.float32).at[:K, :].set(W1.T)

    out = pl.pallas_call(
        _mlp_body,
        out_shape=jax.ShapeDtypeStruct((B, 1), jnp.float32),
    )(x, w1p, b1, g1, be1, W2.T, b2, g2, be2, jnp.reshape(W3, (1, 128)),
      jnp.reshape(b3, (1, 1)))
    return out
